# baseline (speedup 1.0000x reference)
import jax
import jax.numpy as jnp
import numpy as np
from jax.experimental import pallas as pl
from jax.experimental.pallas import tpu as pltpu

_ROWS = 64
_VOCAB = 1_000_000
_BLK = 2048
_NB = (_VOCAB + _BLK - 1) // _BLK

_K0 = np.uint32(0)
_K1 = np.uint32(42)
_KS2 = np.uint32(np.uint32(0x1BD11BDA) ^ _K0 ^ _K1)
_TINY = np.float32(np.finfo(np.float32).tiny)
_MAX_LOGIT = np.float32(40.0)
_NEG_INF = np.float32(-np.inf)
_IBIG = np.int32(2**31 - 1)


def _rotl(x, r):
    return jax.lax.shift_left(x, np.uint32(r)) | jax.lax.shift_right_logical(
        x, np.uint32(32 - r))


def _four_rounds(x0, x1, rots):
    for r in rots:
        x0 = x0 + x1
        x1 = _rotl(x1, r)
        x1 = x0 ^ x1
    return x0, x1


def _gumbel_from_index(flat_idx_u32):
    x0 = jnp.zeros_like(flat_idx_u32) + _K0
    x1 = flat_idx_u32 + _K1
    x0, x1 = _four_rounds(x0, x1, (13, 15, 26, 6))
    x0 = x0 + _K1
    x1 = x1 + _KS2 + np.uint32(1)
    x0, x1 = _four_rounds(x0, x1, (17, 29, 16, 24))
    x0 = x0 + _KS2
    x1 = x1 + _K0 + np.uint32(2)
    x0, x1 = _four_rounds(x0, x1, (13, 15, 26, 6))
    x0 = x0 + _K0
    x1 = x1 + _K1 + np.uint32(3)
    x0, x1 = _four_rounds(x0, x1, (17, 29, 16, 24))
    x0 = x0 + _K1
    x1 = x1 + _KS2 + np.uint32(4)
    x0, x1 = _four_rounds(x0, x1, (13, 15, 26, 6))
    x0 = x0 + _KS2
    x1 = x1 + _K0 + np.uint32(5)
    bits = x0 ^ x1
    fb = jax.lax.shift_right_logical(bits, np.uint32(9)) | np.uint32(0x3F800000)
    f = jax.lax.bitcast_convert_type(fb, jnp.float32) - np.float32(1.0)
    u = jnp.maximum(_TINY, f * (np.float32(1.0) - _TINY) + _TINY)
    return -jnp.log(-jnp.log(u))


def _scan_kernel(x_ref, maxc_ref, candc_ref, candidx_ref):
    j = pl.program_id(0)
    x = x_ref[...]
    col = jax.lax.broadcasted_iota(jnp.int32, (_ROWS, _BLK), 1) + j * _BLK
    valid = col < _VOCAB
    c = jnp.where(valid, jnp.minimum(x, _MAX_LOGIT), _NEG_INF)
    row = jax.lax.broadcasted_iota(jnp.int32, (_ROWS, _BLK), 0)
    flat = (row * _VOCAB + col).astype(jnp.uint32)
    g = _gumbel_from_index(flat)
    q = jnp.where(valid, c + g, _NEG_INF)
    qmax = jnp.max(q, axis=1, keepdims=True)
    idx = jnp.min(jnp.where(q == qmax, col, _IBIG), axis=1, keepdims=True)
    sel = col == idx
    maxc_ref[0, 0, :] = jnp.max(c, axis=1)
    candc_ref[0, 0, :] = jnp.max(jnp.where(sel, c, _NEG_INF), axis=1)
    candidx_ref[0, 0, :] = idx[:, 0]


def _pick_kernel(maxc_ref, candc_ref, candidx_ref, out_ref):
    maxc = maxc_ref[...]
    candc = candc_ref[...]
    candidx = candidx_ref[...]
    m = jnp.max(maxc, axis=0, keepdims=True)
    row = jax.lax.broadcasted_iota(jnp.int32, (_NB, 1, _ROWS), 2)
    flat = (row * _VOCAB + candidx).astype(jnp.uint32)
    g = _gumbel_from_index(flat)
    v = (candc - m) + g
    v = jnp.where(candc > _NEG_INF, v, _NEG_INF)
    vmax = jnp.max(v, axis=0, keepdims=True)
    out_ref[...] = jnp.min(jnp.where(v == vmax, candidx, _IBIG), axis=0)


def kernel(logits):
    maxc, candc, candidx = pl.pallas_call(
        _scan_kernel,
        grid=(_NB,),
        in_specs=[pl.BlockSpec((_ROWS, _BLK), lambda j: (0, j))],
        out_specs=[
            pl.BlockSpec((1, 1, _ROWS), lambda j: (j, 0, 0)),
            pl.BlockSpec((1, 1, _ROWS), lambda j: (j, 0, 0)),
            pl.BlockSpec((1, 1, _ROWS), lambda j: (j, 0, 0)),
        ],
        out_shape=[
            jax.ShapeDtypeStruct((_NB, 1, _ROWS), jnp.float32),
            jax.ShapeDtypeStruct((_NB, 1, _ROWS), jnp.float32),
            jax.ShapeDtypeStruct((_NB, 1, _ROWS), jnp.int32),
        ],
        compiler_params=pltpu.CompilerParams(
            dimension_semantics=("parallel",)),
    )(logits)
    out = pl.pallas_call(
        _pick_kernel,
        out_shape=jax.ShapeDtypeStruct((1, _ROWS), jnp.int32),
    )(maxc, candc, candidx)
    return out.reshape(_ROWS, 1).astype(jnp.int64)


# speedup vs baseline: 1.0371x; 1.0371x over previous
import jax
import jax.numpy as jnp
import numpy as np
from jax.experimental import pallas as pl
from jax.experimental.pallas import tpu as pltpu

_ROWS = 64
_VOCAB = 1_000_000
_BLK = 2048
_NB = (_VOCAB + _BLK - 1) // _BLK

_K0 = np.uint32(0)
_K1 = np.uint32(42)
_KS2 = np.uint32(np.uint32(0x1BD11BDA) ^ _K0 ^ _K1)
_TINY = np.float32(np.finfo(np.float32).tiny)
_MAX_LOGIT = np.float32(40.0)
_NEG_INF = np.float32(-np.inf)
_IBIG = np.int32(2**31 - 1)


def _rotl(x, r):
    return jax.lax.shift_left(x, np.uint32(r)) | jax.lax.shift_right_logical(
        x, np.uint32(32 - r))


def _four_rounds(x0, x1, rots):
    for r in rots:
        x0 = x0 + x1
        x1 = _rotl(x1, r)
        x1 = x0 ^ x1
    return x0, x1


def _gumbel_from_index(flat_idx_u32):
    x0 = jnp.zeros_like(flat_idx_u32) + _K0
    x1 = flat_idx_u32 + _K1
    x0, x1 = _four_rounds(x0, x1, (13, 15, 26, 6))
    x0 = x0 + _K1
    x1 = x1 + _KS2 + np.uint32(1)
    x0, x1 = _four_rounds(x0, x1, (17, 29, 16, 24))
    x0 = x0 + _KS2
    x1 = x1 + _K0 + np.uint32(2)
    x0, x1 = _four_rounds(x0, x1, (13, 15, 26, 6))
    x0 = x0 + _K0
    x1 = x1 + _K1 + np.uint32(3)
    x0, x1 = _four_rounds(x0, x1, (17, 29, 16, 24))
    x0 = x0 + _K1
    x1 = x1 + _KS2 + np.uint32(4)
    x0, x1 = _four_rounds(x0, x1, (13, 15, 26, 6))
    x0 = x0 + _KS2
    x1 = x1 + _K0 + np.uint32(5)
    bits = x0 ^ x1
    fb = jax.lax.shift_right_logical(bits, np.uint32(9)) | np.uint32(0x3F800000)
    f = jax.lax.bitcast_convert_type(fb, jnp.float32) - np.float32(1.0)
    u = jnp.maximum(_TINY, f * (np.float32(1.0) - _TINY) + _TINY)
    return -jnp.log(-jnp.log(u))


_CHUNK = 256
_NCH = _BLK // _CHUNK


def _scan_kernel(x_ref, maxc_ref, candc_ref, candidx_ref):
    j = pl.program_id(0)
    base = j * _BLK
    col_iota = jax.lax.broadcasted_iota(jnp.int32, (_ROWS, _CHUNK), 1)
    row = jax.lax.broadcasted_iota(jnp.int32, (_ROWS, _CHUNK), 0)
    rowoff = row * _VOCAB
    qbest = jnp.full((_ROWS, _CHUNK), _NEG_INF, jnp.float32)
    cbest = jnp.full((_ROWS, _CHUNK), _NEG_INF, jnp.float32)
    ibest = jnp.zeros((_ROWS, _CHUNK), jnp.int32)
    cmax = jnp.full((_ROWS, _CHUNK), _NEG_INF, jnp.float32)
    for k in range(_NCH):
        x = x_ref[:, k * _CHUNK:(k + 1) * _CHUNK]
        col = col_iota + (base + k * _CHUNK)
        valid = col < _VOCAB
        c = jnp.where(valid, jnp.minimum(x, _MAX_LOGIT), _NEG_INF)
        g = _gumbel_from_index((rowoff + col).astype(jnp.uint32))
        q = jnp.where(valid, c + g, _NEG_INF)
        upd = q > qbest
        qbest = jnp.where(upd, q, qbest)
        cbest = jnp.where(upd, c, cbest)
        ibest = jnp.where(upd, col, ibest)
        cmax = jnp.maximum(cmax, c)
    qmax = jnp.max(qbest, axis=1, keepdims=True)
    elig = qbest == qmax
    idxsel = jnp.min(jnp.where(elig, ibest, _IBIG), axis=1, keepdims=True)
    maxc_ref[0, 0, :] = jnp.max(cmax, axis=1)
    candidx_ref[0, 0, :] = idxsel[:, 0]
    candc_ref[0, 0, :] = jnp.max(
        jnp.where(ibest == idxsel, cbest, _NEG_INF), axis=1)


def _pick_kernel(maxc_ref, candc_ref, candidx_ref, out_ref):
    maxc = maxc_ref[...]
    candc = candc_ref[...]
    candidx = candidx_ref[...]
    m = jnp.max(maxc, axis=0, keepdims=True)
    row = jax.lax.broadcasted_iota(jnp.int32, (_NB, 1, _ROWS), 2)
    flat = (row * _VOCAB + candidx).astype(jnp.uint32)
    g = _gumbel_from_index(flat)
    v = (candc - m) + g
    v = jnp.where(candc > _NEG_INF, v, _NEG_INF)
    vmax = jnp.max(v, axis=0, keepdims=True)
    out_ref[...] = jnp.min(jnp.where(v == vmax, candidx, _IBIG), axis=0)


def kernel(logits):
    maxc, candc, candidx = pl.pallas_call(
        _scan_kernel,
        grid=(_NB,),
        in_specs=[pl.BlockSpec((_ROWS, _BLK), lambda j: (0, j))],
        out_specs=[
            pl.BlockSpec((1, 1, _ROWS), lambda j: (j, 0, 0)),
            pl.BlockSpec((1, 1, _ROWS), lambda j: (j, 0, 0)),
            pl.BlockSpec((1, 1, _ROWS), lambda j: (j, 0, 0)),
        ],
        out_shape=[
            jax.ShapeDtypeStruct((_NB, 1, _ROWS), jnp.float32),
            jax.ShapeDtypeStruct((_NB, 1, _ROWS), jnp.float32),
            jax.ShapeDtypeStruct((_NB, 1, _ROWS), jnp.int32),
        ],
        compiler_params=pltpu.CompilerParams(
            dimension_semantics=("parallel",)),
    )(logits)
    out = pl.pallas_call(
        _pick_kernel,
        out_shape=jax.ShapeDtypeStruct((1, _ROWS), jnp.int32),
    )(maxc, candc, candidx)
    return out.reshape(_ROWS, 1).astype(jnp.int64)


# speedup vs baseline: 1.1523x; 1.1110x over previous
import jax
import jax.numpy as jnp
import numpy as np
from jax.experimental import pallas as pl
from jax.experimental.pallas import tpu as pltpu

_ROWS = 64
_VOCAB = 1_000_000
_BLK = 4096
_NB = _VOCAB // _BLK
_COVERED = _NB * _BLK
_TAIL = _VOCAB - _COVERED
_CHUNK = 256
_NCH = _BLK // _CHUNK

_K1 = np.uint32(42)
_KS2 = np.uint32(np.uint32(0x1BD11BDA) ^ np.uint32(42))
_C1 = np.uint32(_KS2 + np.uint32(1))
_C2 = np.uint32(2)
_C3 = np.uint32(45)
_C4 = np.uint32(_KS2 + np.uint32(4))
_C5 = np.uint32(5)
_TINY = np.float32(np.finfo(np.float32).tiny)
_MAX_LOGIT = np.float32(40.0)
_NEG_INF = np.float32(-np.inf)
_IBIG = np.int32(2**31 - 1)


def _rotl(x, r):
    return jax.lax.shift_left(x, np.uint32(r)) | jax.lax.shift_right_logical(
        x, np.uint32(32 - r))


def _sub_round(x0, x1, r):
    x0 = x0 + x1
    x1 = _rotl(x1, r)
    return x0, x0 ^ x1


def _gumbel_from_x1(x1):
    x0 = x1
    x1 = x0 ^ _rotl(x1, 13)
    x0, x1 = _sub_round(x0, x1, 15)
    x0, x1 = _sub_round(x0, x1, 26)
    x0, x1 = _sub_round(x0, x1, 6)
    x0 = x0 + _K1
    x1 = x1 + _C1
    for r in (17, 29, 16, 24):
        x0, x1 = _sub_round(x0, x1, r)
    x0 = x0 + _KS2
    x1 = x1 + _C2
    for r in (13, 15, 26, 6):
        x0, x1 = _sub_round(x0, x1, r)
    x1 = x1 + _C3
    for r in (17, 29, 16, 24):
        x0, x1 = _sub_round(x0, x1, r)
    x0 = x0 + _K1
    x1 = x1 + _C4
    for r in (13, 15, 26, 6):
        x0, x1 = _sub_round(x0, x1, r)
    x0 = x0 + _KS2
    x1 = x1 + _C5
    bits = x0 ^ x1
    fb = jax.lax.shift_right_logical(bits, np.uint32(9)) | np.uint32(0x3F800000)
    f = jax.lax.bitcast_convert_type(fb, jnp.float32) - np.float32(1.0)
    u = f + _TINY
    return -jnp.log(-jnp.log(u))


def _scan_kernel(x_ref, maxc_ref, candc_ref, candidx_ref):
    j = pl.program_id(0)
    lane = jax.lax.broadcasted_iota(jnp.int32, (_ROWS, _CHUNK), 1)
    row = jax.lax.broadcasted_iota(jnp.int32, (_ROWS, _CHUNK), 0)
    rowoff = row * _VOCAB
    cbase = (rowoff + lane + 42).astype(jnp.uint32)
    base_u = (j * _BLK).astype(jnp.uint32)
    qbest = jnp.full((_ROWS, _CHUNK), _NEG_INF, jnp.float32)
    cbest = jnp.full((_ROWS, _CHUNK), _NEG_INF, jnp.float32)
    ibest = jnp.zeros((_ROWS, _CHUNK), jnp.uint32)
    cmax = jnp.full((_ROWS, _CHUNK), _NEG_INF, jnp.float32)
    for k in range(_NCH):
        x1in = cbase + (base_u + np.uint32(k * _CHUNK))
        g = _gumbel_from_x1(x1in)
        c = jnp.minimum(x_ref[:, k * _CHUNK:(k + 1) * _CHUNK], _MAX_LOGIT)
        q = c + g
        upd = q > qbest
        qbest = jnp.where(upd, q, qbest)
        cbest = jnp.where(upd, c, cbest)
        ibest = jnp.where(upd, x1in, ibest)
        cmax = jnp.maximum(cmax, c)
    col = (ibest.astype(jnp.int32) - 42) - rowoff
    qmax = jnp.max(qbest, axis=1, keepdims=True)
    elig = qbest == qmax
    idx = jnp.min(jnp.where(elig, col, _IBIG), axis=1, keepdims=True)
    maxc_ref[0] = jnp.max(cmax, axis=1, keepdims=True)
    candidx_ref[0] = idx
    candc_ref[0] = jnp.max(
        jnp.where(col == idx, cbest, _NEG_INF), axis=1, keepdims=True)


def _pick_kernel(maxc_ref, candc_ref, candidx_ref, tail_ref, out_ref):
    maxc = maxc_ref[...]
    candc = candc_ref[...]
    candidx = candidx_ref[...]
    tailc = jnp.minimum(tail_ref[...], _MAX_LOGIT)
    row_nb = jax.lax.broadcasted_iota(jnp.int32, (_ROWS, _NB), 0)
    row_tl = jax.lax.broadcasted_iota(jnp.int32, (_ROWS, _TAIL), 0)
    tailcol = jax.lax.broadcasted_iota(jnp.int32, (_ROWS, _TAIL), 1) + _COVERED
    m = jnp.maximum(jnp.max(maxc, axis=1, keepdims=True),
                    jnp.max(tailc, axis=1, keepdims=True))
    g_c = _gumbel_from_x1((row_nb * _VOCAB + candidx + 42).astype(jnp.uint32))
    g_t = _gumbel_from_x1((row_tl * _VOCAB + tailcol + 42).astype(jnp.uint32))
    v_c = (candc - m) + g_c
    v_t = (tailc - m) + g_t
    vmax = jnp.maximum(jnp.max(v_c, axis=1, keepdims=True),
                       jnp.max(v_t, axis=1, keepdims=True))
    i_c = jnp.min(jnp.where(v_c == vmax, candidx, _IBIG), axis=1, keepdims=True)
    i_t = jnp.min(jnp.where(v_t == vmax, tailcol, _IBIG), axis=1, keepdims=True)
    out_ref[...] = jnp.minimum(i_c, i_t)


def kernel(logits):
    maxc, candc, candidx = pl.pallas_call(
        _scan_kernel,
        grid=(_NB,),
        in_specs=[pl.BlockSpec((_ROWS, _BLK), lambda j: (0, j))],
        out_specs=[
            pl.BlockSpec((1, _ROWS, 1), lambda j: (j, 0, 0)),
            pl.BlockSpec((1, _ROWS, 1), lambda j: (j, 0, 0)),
            pl.BlockSpec((1, _ROWS, 1), lambda j: (j, 0, 0)),
        ],
        out_shape=[
            jax.ShapeDtypeStruct((_NB, _ROWS, 1), jnp.float32),
            jax.ShapeDtypeStruct((_NB, _ROWS, 1), jnp.float32),
            jax.ShapeDtypeStruct((_NB, _ROWS, 1), jnp.int32),
        ],
        compiler_params=pltpu.CompilerParams(
            dimension_semantics=("arbitrary",)),
    )(logits)
    maxc = maxc.reshape(_NB, _ROWS).T
    candc = candc.reshape(_NB, _ROWS).T
    candidx = candidx.reshape(_NB, _ROWS).T
    tail = jax.lax.slice(logits, (0, _COVERED), (_ROWS, _VOCAB))
    out = pl.pallas_call(
        _pick_kernel,
        out_shape=jax.ShapeDtypeStruct((_ROWS, 1), jnp.int32),
    )(maxc, candc, candidx, tail)
    return out.astype(jnp.int64)
